# Initial kernel scaffold; baseline (speedup 1.0000x reference)
import jax
import jax.numpy as jnp
from jax.experimental import pallas as pl


def kernel(x, edge_index, batch, W1, b1, g1, bt1, W2, b2, g2, bt2, W3, b3, g3, bt3, Wl1, bl1, Wl2, bl2, Wl3, bl3):
    raise NotImplementedError("write your pallas kernel here")


# speedup vs baseline: 5.7504x; 5.7504x over previous
import functools

import jax
import jax.numpy as jnp
from jax import lax
from jax.experimental import pallas as pl
from jax.experimental.pallas import tpu as pltpu
from jax.experimental.pallas import tpu_sc as plsc

N = 10000
E = 160000
F_IN = 16
H = 256
L = 128
B = 64
EPS = 1e-5

NC = 2
NS = 16
NPAD = 10240
EPAD = 163840
BK = 1024
NB = NPAD // BK
RPT = NPAD // NS
CH = 128
EALLOC = EPAD + 2 * CH

def _mesh():
  return plsc.VectorSubcoreMesh(
      core_axis_name="c", subcore_axis_name="s", num_cores=NC, num_subcores=NS)


def _sc_pass(table, src_idx, dst_idx, *, edge_split, gather, init):
  if edge_split:
    ept = EPAD // NC // NS
  else:
    ept = EPAD // NS

  scratch = [
      pltpu.VMEM((CH,), jnp.int32),
      pltpu.VMEM((CH,), jnp.int32),
      pltpu.VMEM((CH, 128), jnp.float32),
      pltpu.VMEM_SHARED((NPAD, 128), jnp.float32),
      pltpu.SemaphoreType.DMA,
  ]

  @functools.partial(
      pl.kernel,
      out_type=jax.ShapeDtypeStruct((NC * NPAD, 128), jnp.float32),
      mesh=_mesh(),
      scratch_types=scratch,
  )
  def k(tab_hbm, src_hbm, dst_hbm, out_hbm, idxs_v, idxd_v, rows_v, acc_sh,
        sem):
    c = lax.axis_index("c")
    s = lax.axis_index("s")

    def fill_rows(val):
      v = jnp.full((16,), val, jnp.float32)

      @pl.loop(0, CH)
      def _(i):
        @pl.loop(0, 128, step=16)
        def _(q):
          rows_v[i, pl.ds(q, 16)] = v

    def zero_acc():
      fill_rows(0.0)

      @pl.loop(0, RPT, step=CH)
      def _(r):
        pltpu.sync_copy(rows_v, acc_sh.at[pl.ds(s * RPT + r, CH)])

    if init == 'self':
      pltpu.sync_copy(tab_hbm.at[pl.ds(c * NPAD + s * RPT, RPT)],
                      acc_sh.at[pl.ds(s * RPT, RPT)])
    elif init == 'core0':
      @pl.when(c == 0)
      def _():
        pltpu.sync_copy(tab_hbm.at[pl.ds(s * RPT, RPT)],
                        acc_sh.at[pl.ds(s * RPT, RPT)])

      @pl.when(c != 0)
      def _():
        zero_acc()
    else:
      zero_acc()

    if not gather:
      fill_rows(1.0)

    plsc.subcore_barrier()

    if edge_split:
      sbase = c * (EPAD // NC) + s * ept
      dbase = c * (EPAD // NC) + s * ept
    else:
      sbase = c * EALLOC + s * ept
      dbase = s * ept

    @pl.loop(0, ept, step=CH)
    def _(j):
      pltpu.sync_copy(dst_hbm.at[pl.ds(dbase + j, CH)], idxd_v)
      if gather:
        pltpu.sync_copy(src_hbm.at[pl.ds(sbase + j, CH)], idxs_v)
        pltpu.async_copy(tab_hbm.at[idxs_v], rows_v, sem).wait()
      pltpu.sync_copy(rows_v, acc_sh.at[idxd_v], add=True)

    plsc.subcore_barrier()
    pltpu.sync_copy(acc_sh.at[pl.ds(s * RPT, RPT)],
                    out_hbm.at[pl.ds(c * NPAD + s * RPT, RPT)])

  return k(table, src_idx, dst_idx)


def _sc_degree(dst_pad):
  dummy = jnp.zeros((8, 128), jnp.float32)
  return _sc_pass(dummy, dst_pad, dst_pad, edge_split=True, gather=False,
                  init='zero')


def _sc_prop128(hs, src2_pad, dst_pad):
  return _sc_pass(hs, src2_pad, dst_pad, edge_split=False, gather=True,
                  init='self')


def _row_mask(i):
  gidx = lax.broadcasted_iota(jnp.int32, (BK, 1), 0) + i * BK
  return gidx < N


def _tc_prep_mm1(degacc, x_pad, W1):
  def body(deg_ref, x_ref, w_ref, dinv_ref, us_ref):
    def blk(i, _):
      r = pl.ds(i * BK, BK)
      r2 = pl.ds(NPAD + i * BK, BK)
      d = deg_ref[r, 0:1] + deg_ref[r2, 0:1] + 1.0
      dv = jnp.where(_row_mask(i), lax.rsqrt(d), 0.0)
      dinv_ref[r, :] = dv
      u = jnp.dot(x_ref[r, :], w_ref[...],
                  preferred_element_type=jnp.float32)
      us = u * dv
      us_ref[r, :] = us[:, 0:128]
      us_ref[r2, :] = us[:, 128:256]
      return 0

    lax.fori_loop(0, NB, blk, 0)

  return pl.pallas_call(
      body,
      out_shape=(jax.ShapeDtypeStruct((NPAD, 1), jnp.float32),
                 jax.ShapeDtypeStruct((2 * NPAD, 128), jnp.float32)),
  )(degacc, x_pad, W1)


def _bn_pass1(acc_ref, dinv_ref, b_ref, z_ref):
  def blk(i, carry):
    s1, s2 = carry
    r = pl.ds(i * BK, BK)
    r2 = pl.ds(NPAD + i * BK, BK)
    dv = dinv_ref[r, :]
    z = jnp.concatenate([acc_ref[r, :] * dv, acc_ref[r2, :] * dv], axis=1)
    z = z + b_ref[...]
    zm = jnp.where(_row_mask(i), z, 0.0)
    z_ref[r, :] = z
    return (s1 + jnp.sum(zm, axis=0, keepdims=True),
            s2 + jnp.sum(zm * zm, axis=0, keepdims=True))

  zeros = jnp.zeros((1, H), jnp.float32)
  return lax.fori_loop(0, NB, blk, (zeros, zeros))


def _bn_coeffs(s1, s2, g_ref, bt_ref):
  mu = s1 * (1.0 / N)
  var = s2 * (1.0 / N) - mu * mu
  scale = g_ref[...] * lax.rsqrt(var + EPS)
  return scale, bt_ref[...] - mu * scale


def _tc_post_mm(acc, dinv, b, g, bt, Wn):
  def body(acc_ref, dinv_ref, b_ref, g_ref, bt_ref, w_ref, out_ref, z_ref):
    s1, s2 = _bn_pass1(acc_ref, dinv_ref, b_ref, z_ref)
    scale, shift = _bn_coeffs(s1, s2, g_ref, bt_ref)

    def blk2(i, _):
      r = pl.ds(i * BK, BK)
      r2 = pl.ds(NPAD + i * BK, BK)
      h = jnp.maximum(z_ref[r, :] * scale + shift, 0.0)
      u = jnp.dot(h, w_ref[...], preferred_element_type=jnp.float32)
      us = u * dinv_ref[r, :]
      out_ref[r, :] = us[:, 0:128]
      out_ref[r2, :] = us[:, 128:256]
      return 0

    lax.fori_loop(0, NB, blk2, 0)

  return pl.pallas_call(
      body,
      out_shape=jax.ShapeDtypeStruct((2 * NPAD, 128), jnp.float32),
      scratch_shapes=[pltpu.VMEM((NPAD, H), jnp.float32)],
  )(acc, dinv, b, g, bt, Wn)


def _tc_post_pool(acc, dinv, b, g, bt, batch_pad, Wl1, bl1, Wl2, bl2, Wl3,
                  bl3):
  def body(acc_ref, dinv_ref, b_ref, g_ref, bt_ref, bt2_ref, w1_ref, b1_ref,
           w2_ref, b2_ref, w3_ref, b3_ref, out_ref, z_ref):
    s1, s2 = _bn_pass1(acc_ref, dinv_ref, b_ref, z_ref)
    scale, shift = _bn_coeffs(s1, s2, g_ref, bt_ref)

    ones = jnp.ones((BK, 1), jnp.float32)
    dn = (((0,), (0,)), ((), ()))

    def blk2(i, carry):
      ps, cnt = carry
      r = pl.ds(i * BK, BK)
      h = jnp.maximum(z_ref[r, :] * scale + shift, 0.0)
      oh = (bt2_ref[r, :] == lax.broadcasted_iota(jnp.int32, (BK, B), 1)
            ).astype(jnp.float32)
      ps = ps + lax.dot_general(oh, h, dn, preferred_element_type=jnp.float32,
                                precision=lax.Precision.HIGHEST)
      cnt = cnt + lax.dot_general(oh, ones, dn,
                                  preferred_element_type=jnp.float32,
                                  precision=lax.Precision.HIGHEST)
      return ps, cnt

    ps, cnt = lax.fori_loop(
        0, NB, blk2,
        (jnp.zeros((B, H), jnp.float32), jnp.zeros((B, 1), jnp.float32)))
    pooled = ps / jnp.maximum(cnt, 1.0)
    z1 = jnp.maximum(
        jnp.dot(pooled, w1_ref[...], preferred_element_type=jnp.float32)
        + b1_ref[...], 0.0)
    z2 = jnp.maximum(
        jnp.dot(z1, w2_ref[...], preferred_element_type=jnp.float32)
        + b2_ref[...], 0.0)
    out_ref[...] = (jnp.dot(z2, w3_ref[...],
                            preferred_element_type=jnp.float32) + b3_ref[...])

  return pl.pallas_call(
      body,
      out_shape=jax.ShapeDtypeStruct((B, 1), jnp.float32),
      scratch_shapes=[pltpu.VMEM((NPAD, H), jnp.float32)],
  )(acc, dinv, b, g, bt, batch_pad, Wl1, bl1, Wl2, bl2, Wl3, bl3)


def kernel(x, edge_index, batch, W1, b1, g1, bt1, W2, b2, g2, bt2,
           W3, b3, g3, bt3, Wl1, bl1, Wl2, bl2, Wl3, bl3):
  src = edge_index[0]
  dst = edge_index[1]

  pad = jnp.full((EALLOC - E,), N, jnp.int32)
  src_pad = jnp.concatenate([src, pad])
  dst_pad = jnp.concatenate([dst, pad])
  src2_pad = jnp.concatenate([src_pad, src_pad + NPAD])

  x_pad = jnp.zeros((NPAD, F_IN), jnp.float32).at[0:N, :].set(x)
  batch_pad = jnp.full((NPAD, 1), B, jnp.int32).at[0:N, 0].set(batch)

  b1r, g1r, bt1r = b1.reshape(1, H), g1.reshape(1, H), bt1.reshape(1, H)
  b2r, g2r, bt2r = b2.reshape(1, H), g2.reshape(1, H), bt2.reshape(1, H)
  b3r, g3r, bt3r = b3.reshape(1, H), g3.reshape(1, H), bt3.reshape(1, H)

  degacc = _sc_degree(dst_pad)
  dinv, us1 = _tc_prep_mm1(degacc, x_pad, W1)

  acc1 = _sc_prop128(us1, src2_pad, dst_pad)
  us2 = _tc_post_mm(acc1, dinv, b1r, g1r, bt1r, W2)

  acc2 = _sc_prop128(us2, src2_pad, dst_pad)
  us3 = _tc_post_mm(acc2, dinv, b2r, g2r, bt2r, W3)

  acc3 = _sc_prop128(us3, src2_pad, dst_pad)
  out = _tc_post_pool(acc3, dinv, b3r, g3r, bt3r, batch_pad,
                      Wl1, bl1.reshape(1, L), Wl2, bl2.reshape(1, L),
                      Wl3, bl3.reshape(1, 1))
  return out.reshape(-1)


# speedup vs baseline: 8.2326x; 1.4317x over previous
import functools

import jax
import jax.numpy as jnp
from jax import lax
from jax.experimental import pallas as pl
from jax.experimental.pallas import tpu as pltpu
from jax.experimental.pallas import tpu_sc as plsc

N = 10000
E = 160000
F_IN = 16
H = 256
L = 128
B = 64
EPS = 1e-5

NC = 2
NS = 16
NPAD = 10240
EPAD = 163840
BK = 1024
NB = NPAD // BK
RPT = NPAD // NS
CH = 80
EALLOC = EPAD + 2 * CH

def _mesh():
  return plsc.VectorSubcoreMesh(
      core_axis_name="c", subcore_axis_name="s", num_cores=NC, num_subcores=NS)


NBUF = 4


def _sc_pass(table, src_idx, dst_idx, *, edge_split, gather, init):
  if edge_split:
    ept = EPAD // NC // NS
  else:
    ept = EPAD // NS
  nch = ept // CH
  tot = ((nch + 2 + NBUF - 1) // NBUF) * NBUF

  scratch = [
      pltpu.VMEM((NBUF, CH), jnp.int32),
      pltpu.VMEM((NBUF, CH), jnp.int32),
      pltpu.VMEM((NBUF, CH, 128), jnp.float32),
      pltpu.VMEM_SHARED((NPAD, 128), jnp.float32),
  ] + [pltpu.SemaphoreType.DMA] * (3 * NBUF)

  @functools.partial(
      pl.kernel,
      out_type=jax.ShapeDtypeStruct((NC * NPAD, 128), jnp.float32),
      mesh=_mesh(),
      scratch_types=scratch,
  )
  def k(tab_hbm, src_hbm, dst_hbm, out_hbm, idxs_v, idxd_v, rows_v, acc_sh,
        *sems):
    sem_idx = sems[0:NBUF]
    sem_gat = sems[NBUF:2 * NBUF]
    sem_sca = sems[2 * NBUF:3 * NBUF]
    c = lax.axis_index("c")
    s = lax.axis_index("s")

    def fill_rows0(val):
      v = jnp.full((16,), val, jnp.float32)

      @pl.loop(0, CH)
      def _(i):
        @pl.loop(0, 128, step=16)
        def _(q):
          rows_v[0, i, pl.ds(q, 16)] = v

    if init == 'self':
      pltpu.sync_copy(tab_hbm.at[pl.ds(c * NPAD + s * RPT, RPT)],
                      acc_sh.at[pl.ds(s * RPT, RPT)])
    else:
      fill_rows0(0.0)

      @pl.loop(0, RPT, step=CH)
      def _(r):
        pltpu.sync_copy(rows_v.at[0], acc_sh.at[pl.ds(s * RPT + r, CH)])

    if not gather:
      fill_rows0(1.0)

    plsc.subcore_barrier()

    if edge_split:
      sbase = c * (EPAD // NC) + s * ept
      dbase = sbase
    else:
      sbase = c * EALLOC + s * ept
      dbase = s * ept

    def idx_issue(g, b):
      pltpu.async_copy(dst_hbm.at[pl.ds(dbase + g * CH, CH)], idxd_v.at[b],
                       sem_idx[b])
      if gather:
        pltpu.async_copy(src_hbm.at[pl.ds(sbase + g * CH, CH)], idxs_v.at[b],
                         sem_idx[b])

    def idx_wait(g, b):
      pltpu.make_async_copy(dst_hbm.at[pl.ds(dbase + g * CH, CH)],
                            idxd_v.at[b], sem_idx[b]).wait()
      if gather:
        pltpu.make_async_copy(src_hbm.at[pl.ds(sbase + g * CH, CH)],
                              idxs_v.at[b], sem_idx[b]).wait()

    def gat_issue(b):
      pltpu.async_copy(tab_hbm.at[idxs_v.at[b]], rows_v.at[b], sem_gat[b])

    def gat_wait(b):
      pltpu.make_async_copy(tab_hbm.at[idxs_v.at[b]], rows_v.at[b],
                            sem_gat[b]).wait()

    def sca_issue(b):
      rb = b if gather else 0
      pltpu.async_copy(rows_v.at[rb], acc_sh.at[idxd_v.at[b]], sem_sca[b],
                       add=True)

    def sca_wait(b):
      rb = b if gather else 0
      pltpu.make_async_copy(rows_v.at[rb], acc_sh.at[idxd_v.at[b]],
                            sem_sca[b]).wait()

    for b in range(NBUF):
      idx_issue(b, b)

    @pl.loop(0, tot, step=NBUF)
    def _(G):
      for kk in range(NBUF):
        g = G + kk
        b = kk
        b1 = (kk - 1) % NBUF
        b2 = (kk + 2) % NBUF

        if gather:
          @pl.when(g < nch)
          def _():
            idx_wait(g, b)
            gat_issue(b)

          @pl.when(jnp.logical_and(g >= 1, g <= nch))
          def _():
            gat_wait(b1)
            sca_issue(b1)

          @pl.when(jnp.logical_and(g >= 2, g <= nch + 1))
          def _():
            sca_wait(b2)

          @pl.when(jnp.logical_and(g >= 2, g + 2 < nch))
          def _():
            idx_issue(g + 2, b2)
        else:
          @pl.when(g < nch)
          def _():
            idx_wait(g, b)
            sca_issue(b)

          @pl.when(jnp.logical_and(g >= 2, g <= nch + 1))
          def _():
            sca_wait(b2)

          @pl.when(jnp.logical_and(g >= 2, g + 2 < nch))
          def _():
            idx_issue(g + 2, b2)

    plsc.subcore_barrier()
    pltpu.sync_copy(acc_sh.at[pl.ds(s * RPT, RPT)],
                    out_hbm.at[pl.ds(c * NPAD + s * RPT, RPT)])

  return k(table, src_idx, dst_idx)


def _sc_degree(dst_pad):
  dummy = jnp.zeros((8, 128), jnp.float32)
  return _sc_pass(dummy, dst_pad, dst_pad, edge_split=True, gather=False,
                  init='zero')


def _sc_prop128(hs, src2_pad, dst_pad):
  return _sc_pass(hs, src2_pad, dst_pad, edge_split=False, gather=True,
                  init='self')


def _row_mask(i):
  gidx = lax.broadcasted_iota(jnp.int32, (BK, 1), 0) + i * BK
  return gidx < N


def _tc_prep_mm1(degacc, x_pad, W1):
  def body(deg_ref, x_ref, w_ref, dinv_ref, us_ref):
    def blk(i, _):
      r = pl.ds(i * BK, BK)
      r2 = pl.ds(NPAD + i * BK, BK)
      d = deg_ref[r, 0:1] + deg_ref[r2, 0:1] + 1.0
      dv = jnp.where(_row_mask(i), lax.rsqrt(d), 0.0)
      dinv_ref[r, :] = dv
      u = jnp.dot(x_ref[r, :], w_ref[...],
                  preferred_element_type=jnp.float32)
      us = u * dv
      us_ref[r, :] = us[:, 0:128]
      us_ref[r2, :] = us[:, 128:256]
      return 0

    lax.fori_loop(0, NB, blk, 0)

  return pl.pallas_call(
      body,
      out_shape=(jax.ShapeDtypeStruct((NPAD, 1), jnp.float32),
                 jax.ShapeDtypeStruct((2 * NPAD, 128), jnp.float32)),
  )(degacc, x_pad, W1)


def _bn_pass1(acc_ref, dinv_ref, b_ref, z_ref):
  def blk(i, carry):
    s1, s2 = carry
    r = pl.ds(i * BK, BK)
    r2 = pl.ds(NPAD + i * BK, BK)
    dv = dinv_ref[r, :]
    z = jnp.concatenate([acc_ref[r, :] * dv, acc_ref[r2, :] * dv], axis=1)
    z = z + b_ref[...]
    zm = jnp.where(_row_mask(i), z, 0.0)
    z_ref[r, :] = z
    return (s1 + jnp.sum(zm, axis=0, keepdims=True),
            s2 + jnp.sum(zm * zm, axis=0, keepdims=True))

  zeros = jnp.zeros((1, H), jnp.float32)
  return lax.fori_loop(0, NB, blk, (zeros, zeros))


def _bn_coeffs(s1, s2, g_ref, bt_ref):
  mu = s1 * (1.0 / N)
  var = s2 * (1.0 / N) - mu * mu
  scale = g_ref[...] * lax.rsqrt(var + EPS)
  return scale, bt_ref[...] - mu * scale


def _tc_post_mm(acc, dinv, b, g, bt, Wn):
  def body(acc_ref, dinv_ref, b_ref, g_ref, bt_ref, w_ref, out_ref, z_ref):
    s1, s2 = _bn_pass1(acc_ref, dinv_ref, b_ref, z_ref)
    scale, shift = _bn_coeffs(s1, s2, g_ref, bt_ref)

    def blk2(i, _):
      r = pl.ds(i * BK, BK)
      r2 = pl.ds(NPAD + i * BK, BK)
      h = jnp.maximum(z_ref[r, :] * scale + shift, 0.0)
      u = jnp.dot(h, w_ref[...], preferred_element_type=jnp.float32)
      us = u * dinv_ref[r, :]
      out_ref[r, :] = us[:, 0:128]
      out_ref[r2, :] = us[:, 128:256]
      return 0

    lax.fori_loop(0, NB, blk2, 0)

  return pl.pallas_call(
      body,
      out_shape=jax.ShapeDtypeStruct((2 * NPAD, 128), jnp.float32),
      scratch_shapes=[pltpu.VMEM((NPAD, H), jnp.float32)],
  )(acc, dinv, b, g, bt, Wn)


def _tc_post_pool(acc, dinv, b, g, bt, batch_pad, Wl1, bl1, Wl2, bl2, Wl3,
                  bl3):
  def body(acc_ref, dinv_ref, b_ref, g_ref, bt_ref, bt2_ref, w1_ref, b1_ref,
           w2_ref, b2_ref, w3_ref, b3_ref, out_ref, z_ref):
    s1, s2 = _bn_pass1(acc_ref, dinv_ref, b_ref, z_ref)
    scale, shift = _bn_coeffs(s1, s2, g_ref, bt_ref)

    ones = jnp.ones((BK, 1), jnp.float32)
    dn = (((0,), (0,)), ((), ()))

    def blk2(i, carry):
      ps, cnt = carry
      r = pl.ds(i * BK, BK)
      h = jnp.maximum(z_ref[r, :] * scale + shift, 0.0)
      oh = (bt2_ref[r, :] == lax.broadcasted_iota(jnp.int32, (BK, B), 1)
            ).astype(jnp.float32)
      ps = ps + lax.dot_general(oh, h, dn, preferred_element_type=jnp.float32,
                                precision=lax.Precision.HIGHEST)
      cnt = cnt + lax.dot_general(oh, ones, dn,
                                  preferred_element_type=jnp.float32,
                                  precision=lax.Precision.HIGHEST)
      return ps, cnt

    ps, cnt = lax.fori_loop(
        0, NB, blk2,
        (jnp.zeros((B, H), jnp.float32), jnp.zeros((B, 1), jnp.float32)))
    pooled = ps / jnp.maximum(cnt, 1.0)
    z1 = jnp.maximum(
        jnp.dot(pooled, w1_ref[...], preferred_element_type=jnp.float32)
        + b1_ref[...], 0.0)
    z2 = jnp.maximum(
        jnp.dot(z1, w2_ref[...], preferred_element_type=jnp.float32)
        + b2_ref[...], 0.0)
    out_ref[...] = (jnp.dot(z2, w3_ref[...],
                            preferred_element_type=jnp.float32) + b3_ref[...])

  return pl.pallas_call(
      body,
      out_shape=jax.ShapeDtypeStruct((B, 1), jnp.float32),
      scratch_shapes=[pltpu.VMEM((NPAD, H), jnp.float32)],
  )(acc, dinv, b, g, bt, batch_pad, Wl1, bl1, Wl2, bl2, Wl3, bl3)


def kernel(x, edge_index, batch, W1, b1, g1, bt1, W2, b2, g2, bt2,
           W3, b3, g3, bt3, Wl1, bl1, Wl2, bl2, Wl3, bl3):
  src = edge_index[0]
  dst = edge_index[1]

  pad = jnp.full((EALLOC - E,), N, jnp.int32)
  src_pad = jnp.concatenate([src, pad])
  dst_pad = jnp.concatenate([dst, pad])
  src2_pad = jnp.concatenate([src_pad, src_pad + NPAD])

  x_pad = jnp.zeros((NPAD, F_IN), jnp.float32).at[0:N, :].set(x)
  batch_pad = jnp.full((NPAD, 1), B, jnp.int32).at[0:N, 0].set(batch)

  b1r, g1r, bt1r = b1.reshape(1, H), g1.reshape(1, H), bt1.reshape(1, H)
  b2r, g2r, bt2r = b2.reshape(1, H), g2.reshape(1, H), bt2.reshape(1, H)
  b3r, g3r, bt3r = b3.reshape(1, H), g3.reshape(1, H), bt3.reshape(1, H)

  degacc = _sc_degree(dst_pad)
  dinv, us1 = _tc_prep_mm1(degacc, x_pad, W1)

  acc1 = _sc_prop128(us1, src2_pad, dst_pad)
  us2 = _tc_post_mm(acc1, dinv, b1r, g1r, bt1r, W2)

  acc2 = _sc_prop128(us2, src2_pad, dst_pad)
  us3 = _tc_post_mm(acc2, dinv, b2r, g2r, bt2r, W3)

  acc3 = _sc_prop128(us3, src2_pad, dst_pad)
  out = _tc_post_pool(acc3, dinv, b3r, g3r, bt3r, batch_pad,
                      Wl1, bl1.reshape(1, L), Wl2, bl2.reshape(1, L),
                      Wl3, bl3.reshape(1, 1))
  return out.reshape(-1)
